# Initial kernel scaffold; baseline (speedup 1.0000x reference)
import jax
import jax.numpy as jnp
from jax.experimental import pallas as pl


def kernel(z_e, codebook):
    raise NotImplementedError("write your pallas kernel here")


# speedup vs baseline: 1.2608x; 1.2608x over previous
import functools

import jax
import jax.numpy as jnp
from jax import lax
from jax.experimental import pallas as pl
from jax.experimental.pallas import tpu as pltpu
from jax.experimental.pallas import tpu_sc as plsc

_NUM_CODES = 1024
_BETA = 0.25
_BN = 1024


def _tc_body(n_total, zf_ref, cb_ref, zsq_ref, esq_ref,
             idx_ref, loss_ref, perp_ref, hist_ref, acc_ref):
    i = pl.program_id(0)
    k = cb_ref.shape[0]
    d = cb_ref.shape[1]

    @pl.when(i == 0)
    def _init():
        hist_ref[...] = jnp.zeros_like(hist_ref)
        acc_ref[0, 0] = 0.0

    zb = zf_ref[...]
    cb = cb_ref[...]
    ze = lax.dot_general(zb, cb, (((1,), (1,)), ((), ())),
                         preferred_element_type=jnp.float32)
    dist = (zsq_ref[...] + esq_ref[...]) - 2.0 * ze
    minval = jnp.min(dist, axis=1, keepdims=True)
    kiota = lax.broadcasted_iota(jnp.int32, dist.shape, 1)
    masked = jnp.where(dist == minval, kiota, k)
    idx = jnp.min(masked, axis=1, keepdims=True)
    idx_ref[...] = idx
    onehot = (masked == idx).astype(jnp.float32)
    hist_ref[...] += jnp.sum(onehot, axis=0, keepdims=True)
    acc_ref[0, 0] += jnp.sum(minval)

    @pl.when(i == pl.num_programs(0) - 1)
    def _fin():
        loss_ref[0, 0] = acc_ref[0, 0] * (1.25 / (n_total * d))
        probs = hist_ref[...] * (1.0 / n_total)
        ent = jnp.sum(probs * jnp.log(probs + 1e-10))
        perp_ref[0, 0] = jnp.exp(-ent)


def _tc_quantize(zf, codebook, z_sq, e_sq):
    n, d = zf.shape
    k = codebook.shape[0]
    grid = n // _BN
    return pl.pallas_call(
        functools.partial(_tc_body, n),
        grid=(grid,),
        in_specs=[
            pl.BlockSpec((_BN, d), lambda i: (i, 0)),
            pl.BlockSpec((k, d), lambda i: (0, 0)),
            pl.BlockSpec((_BN, 1), lambda i: (i, 0)),
            pl.BlockSpec((1, k), lambda i: (0, 0)),
        ],
        out_specs=[
            pl.BlockSpec((_BN, 1), lambda i: (i, 0)),
            pl.BlockSpec(memory_space=pltpu.SMEM),
            pl.BlockSpec(memory_space=pltpu.SMEM),
        ],
        out_shape=[
            jax.ShapeDtypeStruct((n, 1), jnp.int32),
            jax.ShapeDtypeStruct((1, 1), jnp.float32),
            jax.ShapeDtypeStruct((1, 1), jnp.float32),
        ],
        scratch_shapes=[
            pltpu.VMEM((1, k), jnp.float32),
            pltpu.SMEM((1, 1), jnp.float32),
        ],
    )(zf, codebook, z_sq, e_sq)


def _sc_gather(codebook, idx):
    n = idx.shape[0]
    d = codebook.shape[1]
    info = plsc.get_sparse_core_info()
    nw = info.num_cores * info.num_subcores
    bpw = n // nw
    mesh = plsc.VectorSubcoreMesh(core_axis_name="c", subcore_axis_name="s")

    @functools.partial(
        pl.kernel,
        mesh=mesh,
        out_type=jax.ShapeDtypeStruct((n, d), jnp.float32),
        scratch_types=[
            pltpu.VMEM((bpw,), jnp.int32),
            pltpu.VMEM((bpw, d), jnp.float32),
            pltpu.SemaphoreType.DMA,
        ],
        compiler_params=pltpu.CompilerParams(use_tc_tiling_on_sc=False),
    )
    def gather_k(table_hbm, idx_hbm, out_hbm, idx_v, rows_v, sem):
        wid = lax.axis_index("s") * info.num_cores + lax.axis_index("c")
        base = wid * bpw
        pltpu.sync_copy(idx_hbm.at[pl.ds(base, bpw)], idx_v)
        pltpu.async_copy(table_hbm.at[idx_v], rows_v, sem).wait()
        pltpu.sync_copy(rows_v, out_hbm.at[pl.ds(base, bpw)])

    return gather_k(codebook, idx)


def kernel(z_e, codebook):
    b, d, h, w = z_e.shape
    n = b * h * w
    zf = jnp.transpose(z_e, (0, 2, 3, 1)).reshape(n, d)
    z_sq = jnp.sum(zf ** 2, axis=1, keepdims=True)
    e_sq = jnp.sum(codebook ** 2, axis=1).reshape(1, -1)
    idx2, loss, perp = _tc_quantize(zf, codebook, z_sq, e_sq)
    idx = idx2.reshape(n)
    z_q_flat = _sc_gather(codebook, idx)
    z_q = jnp.transpose(z_q_flat.reshape(b, h, w, d), (0, 3, 1, 2))
    return (z_q, loss.reshape(()), idx.reshape(b, h, w), perp.reshape(()))


# speedup vs baseline: 1.4247x; 1.1300x over previous
import functools

import jax
import jax.numpy as jnp
from jax import lax
from jax.experimental import pallas as pl
from jax.experimental.pallas import tpu as pltpu
from jax.experimental.pallas import tpu_sc as plsc

_BN = 8192


def _tc_half_body(zm2_ref, cb_ref, esq_ref, idx_ref, hist_ref, acc_ref):
    k = cb_ref.shape[0]
    zm2 = zm2_ref[...]
    cb = cb_ref[...]
    zsq = jnp.sum(zm2 * zm2, axis=1, keepdims=True) * 0.25
    ze2 = lax.dot_general(zm2, cb, (((1,), (1,)), ((), ())),
                          preferred_element_type=jnp.float32)
    dist = (zsq + esq_ref[...]) + ze2
    minval = jnp.min(dist, axis=1, keepdims=True)
    kiota = lax.broadcasted_iota(jnp.int32, dist.shape, 1)
    masked = jnp.where(dist == minval, kiota, k)
    idx = jnp.min(masked, axis=1, keepdims=True)
    idx_ref[...] = jnp.transpose(idx, (1, 0)).reshape(1, 1, -1)
    onehot = jnp.where(masked == idx, 1.0, 0.0)
    hist_ref[...] = lax.dot_general(
        jnp.ones((1, onehot.shape[0]), jnp.float32), onehot,
        (((1,), (0,)), ((), ())), preferred_element_type=jnp.float32)
    acc_ref[0, 0] = jnp.sum(minval)


def _tc_quantize_half(zm2, codebook, e_sq, half):
    n, d = zm2.shape
    k = codebook.shape[0]
    return pl.pallas_call(
        _tc_half_body,
        grid=(1,),
        in_specs=[
            pl.BlockSpec((_BN, d), lambda i, h=half: (h, 0)),
            pl.BlockSpec((k, d), lambda i: (0, 0)),
            pl.BlockSpec((1, k), lambda i: (0, 0)),
        ],
        out_specs=[
            pl.BlockSpec((1, 1, _BN), lambda i: (0, 0, 0)),
            pl.BlockSpec((1, k), lambda i: (0, 0)),
            pl.BlockSpec(memory_space=pltpu.SMEM),
        ],
        out_shape=[
            jax.ShapeDtypeStruct((1, 1, _BN), jnp.int32),
            jax.ShapeDtypeStruct((1, k), jnp.float32),
            jax.ShapeDtypeStruct((1, 1), jnp.float32),
        ],
    )(zm2, codebook, e_sq)


def _tc_final_body(n_total, d, ha_ref, hb_ref, la_ref, lb_ref,
                   loss_ref, perp_ref):
    loss_ref[0, 0] = (la_ref[0, 0] + lb_ref[0, 0]) * (1.25 / (n_total * d))
    probs = (ha_ref[...] + hb_ref[...]) * (1.0 / n_total)
    ent = jnp.sum(probs * jnp.log(probs + 1e-10))
    perp_ref[0, 0] = jnp.exp(-ent)


def _tc_final(n, d, ha, hb, la, lb):
    k = ha.shape[1]
    return pl.pallas_call(
        functools.partial(_tc_final_body, n, d),
        grid=(1,),
        in_specs=[
            pl.BlockSpec((1, k), lambda i: (0, 0)),
            pl.BlockSpec((1, k), lambda i: (0, 0)),
            pl.BlockSpec(memory_space=pltpu.SMEM),
            pl.BlockSpec(memory_space=pltpu.SMEM),
        ],
        out_specs=[
            pl.BlockSpec(memory_space=pltpu.SMEM),
            pl.BlockSpec(memory_space=pltpu.SMEM),
        ],
        out_shape=[
            jax.ShapeDtypeStruct((1, 1), jnp.float32),
            jax.ShapeDtypeStruct((1, 1), jnp.float32),
        ],
    )(ha, hb, la, lb)


def _sc_gather(codebook, idx):
    n = idx.shape[0]
    d = codebook.shape[1]
    info = plsc.get_sparse_core_info()
    nw = info.num_cores * info.num_subcores
    bpw = n // nw
    mesh = plsc.VectorSubcoreMesh(core_axis_name="c", subcore_axis_name="s")

    @functools.partial(
        pl.kernel,
        mesh=mesh,
        out_type=jax.ShapeDtypeStruct((n, d), jnp.float32),
        scratch_types=[
            pltpu.VMEM((bpw,), jnp.int32),
            pltpu.VMEM((bpw, d), jnp.float32),
            pltpu.SemaphoreType.DMA,
        ],
        compiler_params=pltpu.CompilerParams(use_tc_tiling_on_sc=False),
    )
    def gather_k(table_hbm, idx_hbm, out_hbm, idx_v, rows_v, sem):
        wid = lax.axis_index("s") * info.num_cores + lax.axis_index("c")
        base = wid * bpw
        pltpu.sync_copy(idx_hbm.at[pl.ds(base, bpw)], idx_v)
        pltpu.async_copy(table_hbm.at[idx_v], rows_v, sem).wait()
        pltpu.sync_copy(rows_v, out_hbm.at[pl.ds(base, bpw)])

    return gather_k(codebook, idx)


def kernel(z_e, codebook):
    b, d, h, w = z_e.shape
    n = b * h * w
    bh = _BN // (h * w)
    zm2 = jnp.transpose(z_e, (0, 2, 3, 1)).reshape(n, d) * (-2.0)
    e_sq = jnp.sum(codebook ** 2, axis=1).reshape(1, -1)
    ia, ha, la = _tc_quantize_half(zm2, codebook, e_sq, 0)
    ib, hb, lb = _tc_quantize_half(zm2, codebook, e_sq, 1)
    zqa = _sc_gather(codebook, ia.reshape(_BN))
    zqb = _sc_gather(codebook, ib.reshape(_BN))
    loss, perp = _tc_final(n, d, ha, hb, la, lb)
    za = jnp.transpose(zqa.reshape(bh, h, w, d), (0, 3, 1, 2))
    zb = jnp.transpose(zqb.reshape(bh, h, w, d), (0, 3, 1, 2))
    z_q = jnp.concatenate([za, zb], axis=0)
    idx = jnp.concatenate([ia.reshape(-1), ib.reshape(-1)]).reshape(b, h, w)
    return (z_q, loss.reshape(()), idx, perp.reshape(()))


# speedup vs baseline: 1.4463x; 1.0152x over previous
import functools

import jax
import jax.numpy as jnp
from jax import lax
from jax.experimental import pallas as pl
from jax.experimental.pallas import tpu as pltpu
from jax.experimental.pallas import tpu_sc as plsc

_BN = 8192


def _tc_half_body(zm2_ref, cb_ref, esq_ref, idx_ref, hist_ref, acc_ref):
    k = cb_ref.shape[0]
    zm2 = zm2_ref[...]
    cb = cb_ref[...]
    zsq = jnp.sum(zm2 * zm2, axis=1, keepdims=True) * 0.25
    ze2 = lax.dot_general(zm2, cb, (((1,), (1,)), ((), ())),
                          preferred_element_type=jnp.float32)
    dist = (zsq + esq_ref[...]) + ze2
    minval = jnp.min(dist, axis=1, keepdims=True)
    kiota = lax.broadcasted_iota(jnp.int32, dist.shape, 1)
    masked = jnp.where(dist == minval, kiota, k)
    idx = jnp.min(masked, axis=1, keepdims=True)
    idx_ref[...] = jnp.transpose(idx, (1, 0)).reshape(1, 1, -1)
    onehot = jnp.where(masked == idx, 1.0, 0.0)
    hist_ref[...] = lax.dot_general(
        jnp.ones((1, onehot.shape[0]), jnp.float32), onehot,
        (((1,), (0,)), ((), ())), preferred_element_type=jnp.float32)
    acc_ref[0, 0] = jnp.sum(minval)


def _tc_quantize_half(zm2, codebook, e_sq, half):
    n, d = zm2.shape
    k = codebook.shape[0]
    return pl.pallas_call(
        _tc_half_body,
        grid=(1,),
        in_specs=[
            pl.BlockSpec((_BN, d), lambda i, h=half: (h, 0)),
            pl.BlockSpec((k, d), lambda i: (0, 0)),
            pl.BlockSpec((1, k), lambda i: (0, 0)),
        ],
        out_specs=[
            pl.BlockSpec((1, 1, _BN), lambda i: (0, 0, 0)),
            pl.BlockSpec((1, k), lambda i: (0, 0)),
            pl.BlockSpec(memory_space=pltpu.SMEM),
        ],
        out_shape=[
            jax.ShapeDtypeStruct((1, 1, _BN), jnp.int32),
            jax.ShapeDtypeStruct((1, k), jnp.float32),
            jax.ShapeDtypeStruct((1, 1), jnp.float32),
        ],
    )(zm2, codebook, e_sq)


def _tc_final_body(n_total, d, ha_ref, hb_ref, la_ref, lb_ref,
                   loss_ref, perp_ref):
    loss_ref[0, 0] = (la_ref[0, 0] + lb_ref[0, 0]) * (1.25 / (n_total * d))
    probs = (ha_ref[...] + hb_ref[...]) * (1.0 / n_total)
    ent = jnp.sum(probs * jnp.log(probs + 1e-10))
    perp_ref[0, 0] = jnp.exp(-ent)


def _tc_final(n, d, ha, hb, la, lb):
    k = ha.shape[1]
    return pl.pallas_call(
        functools.partial(_tc_final_body, n, d),
        grid=(1,),
        in_specs=[
            pl.BlockSpec((1, k), lambda i: (0, 0)),
            pl.BlockSpec((1, k), lambda i: (0, 0)),
            pl.BlockSpec(memory_space=pltpu.SMEM),
            pl.BlockSpec(memory_space=pltpu.SMEM),
        ],
        out_specs=[
            pl.BlockSpec(memory_space=pltpu.SMEM),
            pl.BlockSpec(memory_space=pltpu.SMEM),
        ],
        out_shape=[
            jax.ShapeDtypeStruct((1, 1), jnp.float32),
            jax.ShapeDtypeStruct((1, 1), jnp.float32),
        ],
    )(ha, hb, la, lb)


def _sc_gather(codebook, idx):
    n = idx.shape[0]
    d = codebook.shape[1]
    info = plsc.get_sparse_core_info()
    nw = info.num_cores * info.num_subcores
    bpw = n // nw
    mesh = plsc.VectorSubcoreMesh(core_axis_name="c", subcore_axis_name="s")

    @functools.partial(
        pl.kernel,
        mesh=mesh,
        out_type=jax.ShapeDtypeStruct((n, d), jnp.float32),
        scratch_types=[
            pltpu.VMEM((bpw,), jnp.int32),
            pltpu.VMEM((bpw, d), jnp.float32),
            pltpu.SemaphoreType.DMA,
        ],
        compiler_params=pltpu.CompilerParams(use_tc_tiling_on_sc=False),
    )
    def gather_k(table_hbm, idx_hbm, out_hbm, idx_v, rows_v, sem):
        wid = lax.axis_index("s") * info.num_cores + lax.axis_index("c")
        base = wid * bpw
        pltpu.sync_copy(idx_hbm.at[pl.ds(base, bpw)], idx_v)
        pltpu.async_copy(table_hbm.at[idx_v], rows_v, sem).wait()
        pltpu.sync_copy(rows_v, out_hbm.at[pl.ds(base, bpw)])

    return gather_k(codebook, idx)


def kernel(z_e, codebook):
    b, d, h, w = z_e.shape
    n = b * h * w
    bh = _BN // (h * w)
    zm2 = jnp.transpose(z_e, (0, 2, 3, 1)).reshape(n, d) * (-2.0)
    e_sq = jnp.sum(codebook ** 2, axis=1).reshape(1, -1)
    ia, ha, la = _tc_quantize_half(zm2, codebook, e_sq, 0)
    ib, hb, lb = _tc_quantize_half(zm2, codebook, e_sq, 1)
    zqa = _sc_gather(codebook, ia.reshape(_BN))
    zqb = _sc_gather(codebook, ib.reshape(_BN))
    loss, perp = _tc_final(n, d, ha, hb, la, lb)
    z_q_flat = jnp.concatenate([zqa, zqb], axis=0)
    z_q = jnp.transpose(z_q_flat.reshape(b, h, w, d), (0, 3, 1, 2))
    idx = jnp.concatenate([ia.reshape(-1), ib.reshape(-1)]).reshape(b, h, w)
    return (z_q, loss.reshape(()), idx, perp.reshape(()))


# speedup vs baseline: 1.5472x; 1.0698x over previous
import functools

import jax
import jax.numpy as jnp
from jax import lax
from jax.experimental import pallas as pl
from jax.experimental.pallas import tpu as pltpu
from jax.experimental.pallas import tpu_sc as plsc

_BN = 8192


def _tc_body(n_total, zm2_ref, cb_ref, esq_ref,
             idx_ref, loss_ref, perp_ref, hist_ref, acc_ref):
    i = pl.program_id(0)
    k = cb_ref.shape[0]
    d = cb_ref.shape[1]

    @pl.when(i == 0)
    def _init():
        hist_ref[...] = jnp.zeros_like(hist_ref)
        acc_ref[0, 0] = 0.0

    zm2 = zm2_ref[...]
    cb = cb_ref[...]
    zsq = jnp.sum(zm2 * zm2, axis=1, keepdims=True) * 0.25
    ze2 = lax.dot_general(zm2, cb, (((1,), (1,)), ((), ())),
                          preferred_element_type=jnp.float32)
    dist = (zsq + esq_ref[...]) + ze2
    minval = jnp.min(dist, axis=1, keepdims=True)
    kiota = lax.broadcasted_iota(jnp.int32, dist.shape, 1)
    masked = jnp.where(dist == minval, kiota, k)
    idx = jnp.min(masked, axis=1, keepdims=True)
    idx_ref[...] = jnp.transpose(idx, (1, 0)).reshape(1, 1, -1)
    onehot = jnp.where(masked == idx, 1.0, 0.0)
    hist_ref[...] += lax.dot_general(
        jnp.ones((1, onehot.shape[0]), jnp.float32), onehot,
        (((1,), (0,)), ((), ())), preferred_element_type=jnp.float32)
    acc_ref[0, 0] += jnp.sum(minval)

    @pl.when(i == pl.num_programs(0) - 1)
    def _fin():
        loss_ref[0, 0] = acc_ref[0, 0] * (1.25 / (n_total * d))
        probs = hist_ref[...] * (1.0 / n_total)
        ent = jnp.sum(probs * jnp.log(probs + 1e-10))
        perp_ref[0, 0] = jnp.exp(-ent)


def _tc_quantize(zm2, codebook, e_sq):
    n, d = zm2.shape
    k = codebook.shape[0]
    grid = n // _BN
    return pl.pallas_call(
        functools.partial(_tc_body, n),
        grid=(grid,),
        in_specs=[
            pl.BlockSpec((_BN, d), lambda i: (i, 0)),
            pl.BlockSpec((k, d), lambda i: (0, 0)),
            pl.BlockSpec((1, k), lambda i: (0, 0)),
        ],
        out_specs=[
            pl.BlockSpec((1, 1, _BN), lambda i: (i, 0, 0)),
            pl.BlockSpec(memory_space=pltpu.SMEM),
            pl.BlockSpec(memory_space=pltpu.SMEM),
        ],
        out_shape=[
            jax.ShapeDtypeStruct((grid, 1, _BN), jnp.int32),
            jax.ShapeDtypeStruct((1, 1), jnp.float32),
            jax.ShapeDtypeStruct((1, 1), jnp.float32),
        ],
        scratch_shapes=[
            pltpu.VMEM((1, k), jnp.float32),
            pltpu.SMEM((1, 1), jnp.float32),
        ],
    )(zm2, codebook, e_sq)


def _sc_gather(codebook, idx):
    n = idx.shape[0]
    d = codebook.shape[1]
    info = plsc.get_sparse_core_info()
    nw = info.num_cores * info.num_subcores
    bpw = n // nw
    mesh = plsc.VectorSubcoreMesh(core_axis_name="c", subcore_axis_name="s")

    @functools.partial(
        pl.kernel,
        mesh=mesh,
        out_type=jax.ShapeDtypeStruct((n, d), jnp.float32),
        scratch_types=[
            pltpu.VMEM((bpw,), jnp.int32),
            pltpu.VMEM((bpw, d), jnp.float32),
            pltpu.SemaphoreType.DMA,
        ],
        compiler_params=pltpu.CompilerParams(use_tc_tiling_on_sc=False),
    )
    def gather_k(table_hbm, idx_hbm, out_hbm, idx_v, rows_v, sem):
        wid = lax.axis_index("s") * info.num_cores + lax.axis_index("c")
        base = wid * bpw
        pltpu.sync_copy(idx_hbm.at[pl.ds(base, bpw)], idx_v)
        pltpu.async_copy(table_hbm.at[idx_v], rows_v, sem).wait()
        pltpu.sync_copy(rows_v, out_hbm.at[pl.ds(base, bpw)])

    return gather_k(codebook, idx)


def kernel(z_e, codebook):
    b, d, h, w = z_e.shape
    n = b * h * w
    zm2 = jnp.transpose(z_e, (0, 2, 3, 1)).reshape(n, d) * (-2.0)
    e_sq = jnp.sum(codebook ** 2, axis=1).reshape(1, -1)
    idx2d, loss, perp = _tc_quantize(zm2, codebook, e_sq)
    idx = idx2d.reshape(n)
    z_q_flat = _sc_gather(codebook, idx)
    z_q = jnp.transpose(z_q_flat.reshape(b, h, w, d), (0, 3, 1, 2))
    return (z_q, loss.reshape(()), idx.reshape(b, h, w), perp.reshape(()))
